# Initial kernel scaffold; baseline (speedup 1.0000x reference)
import jax
import jax.numpy as jnp
from jax.experimental import pallas as pl


def kernel(class_logit, box_regression, proposal):
    raise NotImplementedError("write your pallas kernel here")


# speedup vs baseline: 4.0481x; 4.0481x over previous
import math

import jax
import jax.numpy as jnp
from jax import lax
from jax.experimental import pallas as pl

_N = 20000
_NUM_CLASSES = 21
_NC = _NUM_CLASSES - 1
_IMG_H, _IMG_W = 800.0, 1216.0
_SCORE_THRESH = 0.05
_NMS_THRESH = 0.5
_NUM_DETECTIONS = 100
_MIN_SIZE = 1.0
_K_PRE = 1000
_KP = 1024
_B = 128
_BBOX_XFORM_CLIP = math.log(1000.0 / 16.0)


def _prep_kernel(l_ref, dx_ref, dy_ref, dw_ref, dh_ref, p_ref,
                 s_ref, x1_ref, y1_ref, x2_ref, y2_ref):
    l = l_ref[...]
    m = jnp.max(l, axis=0, keepdims=True)
    e = jnp.exp(l - m)
    denom = jnp.sum(e, axis=0, keepdims=True)
    score = e[1:, :] / denom

    px1 = p_ref[0:1, :]
    py1 = p_ref[1:2, :]
    px2 = p_ref[2:3, :]
    py2 = p_ref[3:4, :]
    w = px2 - px1
    h = py2 - py1
    cx = px1 + 0.5 * w
    cy = py1 + 0.5 * h

    dx = dx_ref[...] / 10.0
    dy = dy_ref[...] / 10.0
    dw = jnp.minimum(dw_ref[...] / 5.0, _BBOX_XFORM_CLIP)
    dh = jnp.minimum(dh_ref[...] / 5.0, _BBOX_XFORM_CLIP)

    pcx = dx * w + cx
    pcy = dy * h + cy
    pw = jnp.exp(dw) * w
    ph = jnp.exp(dh) * h

    x1 = jnp.clip(pcx - 0.5 * pw, 0.0, _IMG_W)
    y1 = jnp.clip(pcy - 0.5 * ph, 0.0, _IMG_H)
    x2 = jnp.clip(pcx + 0.5 * pw, 0.0, _IMG_W)
    y2 = jnp.clip(pcy + 0.5 * ph, 0.0, _IMG_H)

    bw = x2 - x1
    bh = y2 - y1
    valid = (score >= _SCORE_THRESH) & (bw >= _MIN_SIZE) & (bh >= _MIN_SIZE)
    s_ref[...] = jnp.where(valid, score, -1.0)
    x1_ref[...] = x1
    y1_ref[...] = y1
    x2_ref[...] = x2
    y2_ref[...] = y2


def _nms_kernel(s_ref, x1r_ref, y1r_ref, x2r_ref, y2r_ref,
                x1c_ref, y1c_ref, x2c_ref, y2c_ref, s2_ref):
    s_row = s_ref[0]
    x1r = x1r_ref[0]
    y1r = y1r_ref[0]
    x2r = x2r_ref[0]
    y2r = y2r_ref[0]
    x1c = x1c_ref[0]
    y1c = y1c_ref[0]
    x2c = x2c_ref[0]
    y2c = y2c_ref[0]

    area_r = (x2r - x1r) * (y2r - y1r)
    area_c = (x2c - x1c) * (y2c - y1c)
    ltx = jnp.maximum(x1c, x1r)
    lty = jnp.maximum(y1c, y1r)
    rbx = jnp.minimum(x2c, x2r)
    rby = jnp.minimum(y2c, y2r)
    iw = jnp.clip(rbx - ltx, 0.0, None)
    ih = jnp.clip(rby - lty, 0.0, None)
    inter = iw * ih
    union = area_c + area_r - inter
    iou = inter / jnp.maximum(union, 1e-9)
    sup = (iou > _NMS_THRESH).astype(jnp.float32)

    validf = (s_row > 0.0).astype(jnp.float32)

    ii = lax.broadcasted_iota(jnp.int32, (_B, _B), 0)
    jj = lax.broadcasted_iota(jnp.int32, (_B, _B), 1)
    tri = (ii < jj).astype(jnp.float32)

    supp_cnt = jnp.zeros((1, _KP), jnp.float32)
    kept_blocks = []
    for b in range(_KP // _B):
        lo = b * _B
        d = sup[lo:lo + _B, lo:lo + _B] * tri
        vblk = validf[:, lo:lo + _B] * (supp_cnt[:, lo:lo + _B] == 0.0)

        def body(st):
            k, _, it = st
            cnt = jnp.dot(k, d, preferred_element_type=jnp.float32)
            knew = vblk * (cnt == 0.0)
            return knew, k, it + 1

        def cond(st):
            k, prev, it = st
            return jnp.logical_and(it < _B, jnp.any(k != prev))

        k0 = vblk
        kfin, _, _ = lax.while_loop(cond, body, (k0, -jnp.ones_like(k0), 0))
        kept_blocks.append(kfin)
        supp_cnt = supp_cnt + jnp.dot(kfin, sup[lo:lo + _B, :],
                                      preferred_element_type=jnp.float32)

    kept = jnp.concatenate(kept_blocks, axis=1)
    s2_ref[...] = jnp.where(kept > 0.0, s_row, -1.0)[None]


def kernel(class_logit, box_regression, proposal):
    lT = class_logit.T
    deltas = box_regression.reshape(_N, _NUM_CLASSES, 4)[:, 1:, :]
    dT = deltas.transpose(1, 2, 0)
    pT = proposal.T

    out_sd = [jax.ShapeDtypeStruct((_NC, _N), jnp.float32)] * 5
    s, bx1, by1, bx2, by2 = pl.pallas_call(
        _prep_kernel,
        out_shape=out_sd,
    )(lT, dT[:, 0], dT[:, 1], dT[:, 2], dT[:, 3], pT)

    top_s, top_i = lax.top_k(s, _K_PRE)
    gx1 = jnp.take_along_axis(bx1, top_i, axis=1)
    gy1 = jnp.take_along_axis(by1, top_i, axis=1)
    gx2 = jnp.take_along_axis(bx2, top_i, axis=1)
    gy2 = jnp.take_along_axis(by2, top_i, axis=1)

    pad = _KP - _K_PRE
    sp = jnp.pad(top_s, ((0, 0), (0, pad)), constant_values=-1.0)
    px1 = jnp.pad(gx1, ((0, 0), (0, pad)))
    py1 = jnp.pad(gy1, ((0, 0), (0, pad)))
    px2 = jnp.pad(gx2, ((0, 0), (0, pad)))
    py2 = jnp.pad(gy2, ((0, 0), (0, pad)))

    row = lambda a: a.reshape(_NC, 1, _KP)
    col = lambda a: a.reshape(_NC, _KP, 1)
    rspec = pl.BlockSpec((1, 1, _KP), lambda c: (c, 0, 0))
    cspec = pl.BlockSpec((1, _KP, 1), lambda c: (c, 0, 0))

    s2 = pl.pallas_call(
        _nms_kernel,
        grid=(_NC,),
        in_specs=[rspec, rspec, rspec, rspec, rspec,
                  cspec, cspec, cspec, cspec],
        out_specs=rspec,
        out_shape=jax.ShapeDtypeStruct((_NC, 1, _KP), jnp.float32),
    )(row(sp), row(px1), row(py1), row(px2), row(py2),
      col(px1), col(py1), col(px2), col(py2))

    s2 = s2.reshape(_NC, _KP)
    det_s, det_i = lax.top_k(s2, _NUM_DETECTIONS)
    dx1 = jnp.take_along_axis(px1, det_i, axis=1)
    dy1 = jnp.take_along_axis(py1, det_i, axis=1)
    dx2 = jnp.take_along_axis(px2, det_i, axis=1)
    dy2 = jnp.take_along_axis(py2, det_i, axis=1)
    det_box = jnp.stack([dx1, dy1, dx2, dy2], axis=-1)

    det_valid = det_s > 0.0
    det_box = jnp.where(det_valid[:, :, None], det_box, 0.0)
    det_s = jnp.where(det_valid, det_s, 0.0)
    labels = jnp.broadcast_to(jnp.arange(1, _NUM_CLASSES)[:, None],
                              (_NC, _NUM_DETECTIONS))
    labels = jnp.where(det_valid, labels, 0).astype(jnp.float32)
    det = jnp.concatenate([det_box.reshape(-1, 4),
                           det_s.reshape(-1, 1),
                           labels.reshape(-1, 1)], axis=1)
    return det


# speedup vs baseline: 4.9491x; 1.2226x over previous
import math

import jax
import jax.numpy as jnp
from jax import lax
from jax.experimental import pallas as pl

_N = 20000
_NUM_CLASSES = 21
_NC = _NUM_CLASSES - 1
_IMG_H, _IMG_W = 800.0, 1216.0
_SCORE_THRESH = 0.05
_NMS_THRESH = 0.5
_NUM_DETECTIONS = 100
_MIN_SIZE = 1.0
_K_PRE = 1000
_KP = 1024
_B = 128
_BBOX_XFORM_CLIP = math.log(1000.0 / 16.0)


def _prep_kernel(l_ref, dx_ref, dy_ref, dw_ref, dh_ref, p_ref,
                 s_ref, x1_ref, y1_ref, x2_ref, y2_ref, t_ref):
    l = l_ref[...]
    m = jnp.max(l, axis=0, keepdims=True)
    e = jnp.exp(l - m)
    denom = jnp.sum(e, axis=0, keepdims=True)
    score = e[1:, :] / denom

    px1 = p_ref[0:1, :]
    py1 = p_ref[1:2, :]
    px2 = p_ref[2:3, :]
    py2 = p_ref[3:4, :]
    w = px2 - px1
    h = py2 - py1
    cx = px1 + 0.5 * w
    cy = py1 + 0.5 * h

    dx = dx_ref[...] / 10.0
    dy = dy_ref[...] / 10.0
    dw = jnp.minimum(dw_ref[...] / 5.0, _BBOX_XFORM_CLIP)
    dh = jnp.minimum(dh_ref[...] / 5.0, _BBOX_XFORM_CLIP)

    pcx = dx * w + cx
    pcy = dy * h + cy
    pw = jnp.exp(dw) * w
    ph = jnp.exp(dh) * h

    x1 = jnp.clip(pcx - 0.5 * pw, 0.0, _IMG_W)
    y1 = jnp.clip(pcy - 0.5 * ph, 0.0, _IMG_H)
    x2 = jnp.clip(pcx + 0.5 * pw, 0.0, _IMG_W)
    y2 = jnp.clip(pcy + 0.5 * ph, 0.0, _IMG_H)

    bw = x2 - x1
    bh = y2 - y1
    valid = (score >= _SCORE_THRESH) & (bw >= _MIN_SIZE) & (bh >= _MIN_SIZE)
    s = jnp.where(valid, score, -1.0)
    s_ref[...] = s
    x1_ref[...] = x1
    y1_ref[...] = y1
    x2_ref[...] = x2
    y2_ref[...] = y2

    keys = jnp.where(s < 0.0, jnp.int32(-1),
                     lax.bitcast_convert_type(s, jnp.int32))
    lo0 = jnp.full((_NC, 1), -1, jnp.int32)
    hi0 = jnp.full((_NC, 1), 0x3F800001, jnp.int32)

    def bs_body(_, st):
        lo, hi = st
        mid = lo + (hi - lo) // 2
        cnt = jnp.sum((keys >= mid).astype(jnp.int32), axis=1, keepdims=True)
        pred = cnt >= _K_PRE
        return jnp.where(pred, mid, lo), jnp.where(pred, hi, mid)

    lo, hi = lax.fori_loop(0, 31, bs_body, (lo0, hi0))
    t_ref[...] = lo


def _nms_kernel(s_ref, x1r_ref, y1r_ref, x2r_ref, y2r_ref,
                x1c_ref, y1c_ref, x2c_ref, y2c_ref, s2_ref):
    s_row = s_ref[0]
    x1r = x1r_ref[0]
    y1r = y1r_ref[0]
    x2r = x2r_ref[0]
    y2r = y2r_ref[0]
    x1c = x1c_ref[0]
    y1c = y1c_ref[0]
    x2c = x2c_ref[0]
    y2c = y2c_ref[0]

    area_r = (x2r - x1r) * (y2r - y1r)
    area_c = (x2c - x1c) * (y2c - y1c)
    ltx = jnp.maximum(x1c, x1r)
    lty = jnp.maximum(y1c, y1r)
    rbx = jnp.minimum(x2c, x2r)
    rby = jnp.minimum(y2c, y2r)
    iw = jnp.clip(rbx - ltx, 0.0, None)
    ih = jnp.clip(rby - lty, 0.0, None)
    inter = iw * ih
    union = area_c + area_r - inter
    iou = inter / jnp.maximum(union, 1e-9)
    sup = (iou > _NMS_THRESH).astype(jnp.float32)

    validf = (s_row > 0.0).astype(jnp.float32)

    ii = lax.broadcasted_iota(jnp.int32, (_B, _B), 0)
    jj = lax.broadcasted_iota(jnp.int32, (_B, _B), 1)
    tri = (ii < jj).astype(jnp.float32)

    supp_cnt = jnp.zeros((1, _KP), jnp.float32)
    kept_blocks = []
    for b in range(_KP // _B):
        lo = b * _B
        d = sup[lo:lo + _B, lo:lo + _B] * tri
        vblk = validf[:, lo:lo + _B] * (supp_cnt[:, lo:lo + _B] == 0.0)

        def body(st):
            k, _, it = st
            cnt = jnp.dot(k, d, preferred_element_type=jnp.float32)
            knew = vblk * (cnt == 0.0)
            return knew, k, it + 1

        def cond(st):
            k, prev, it = st
            return jnp.logical_and(it < _B, jnp.any(k != prev))

        k0 = vblk
        kfin, _, _ = lax.while_loop(cond, body, (k0, -jnp.ones_like(k0), 0))
        kept_blocks.append(kfin)
        supp_cnt = supp_cnt + jnp.dot(kfin, sup[lo:lo + _B, :],
                                      preferred_element_type=jnp.float32)

    kept = jnp.concatenate(kept_blocks, axis=1)
    s2_ref[...] = jnp.where(kept > 0.0, s_row, -1.0)[None]


def kernel(class_logit, box_regression, proposal):
    lT = class_logit.T
    deltas = box_regression.reshape(_N, _NUM_CLASSES, 4)[:, 1:, :]
    dT = deltas.transpose(1, 2, 0)
    pT = proposal.T

    out_sd = [jax.ShapeDtypeStruct((_NC, _N), jnp.float32)] * 5 + [
        jax.ShapeDtypeStruct((_NC, 1), jnp.int32)]
    s, bx1, by1, bx2, by2, tkey = pl.pallas_call(
        _prep_kernel,
        out_shape=out_sd,
    )(lT, dT[:, 0], dT[:, 1], dT[:, 2], dT[:, 3], pT)

    keys = jnp.where(s < 0.0, jnp.int32(-1),
                     lax.bitcast_convert_type(s, jnp.int32))
    mask_gt = keys > tkey
    mask_eq = keys == tkey
    cnt_gt = jnp.sum(mask_gt.astype(jnp.int32), axis=1, keepdims=True)
    cnt_eq = jnp.sum(mask_eq.astype(jnp.int32), axis=1, keepdims=True)
    nz = jax.vmap(lambda m: jnp.nonzero(m, size=_KP, fill_value=0)[0])
    idx_gt = nz(mask_gt).astype(jnp.int32)
    idx_eq = nz(mask_eq).astype(jnp.int32)
    slot = jnp.arange(_KP, dtype=jnp.int32)[None]
    cand_idx = jnp.concatenate([idx_gt, idx_eq], axis=1)
    cand_ok = jnp.concatenate([slot < cnt_gt, slot < cnt_eq], axis=1)
    cand_s = jnp.where(cand_ok,
                       jnp.take_along_axis(s, cand_idx, axis=1), -2.0)
    neg_sorted, idx_sorted = lax.sort((-cand_s, cand_idx), num_keys=2)
    top_s = -neg_sorted[:, :_K_PRE]
    top_i = idx_sorted[:, :_K_PRE]
    gx1 = jnp.take_along_axis(bx1, top_i, axis=1)
    gy1 = jnp.take_along_axis(by1, top_i, axis=1)
    gx2 = jnp.take_along_axis(bx2, top_i, axis=1)
    gy2 = jnp.take_along_axis(by2, top_i, axis=1)

    pad = _KP - _K_PRE
    sp = jnp.pad(top_s, ((0, 0), (0, pad)), constant_values=-1.0)
    px1 = jnp.pad(gx1, ((0, 0), (0, pad)))
    py1 = jnp.pad(gy1, ((0, 0), (0, pad)))
    px2 = jnp.pad(gx2, ((0, 0), (0, pad)))
    py2 = jnp.pad(gy2, ((0, 0), (0, pad)))

    row = lambda a: a.reshape(_NC, 1, _KP)
    col = lambda a: a.reshape(_NC, _KP, 1)
    rspec = pl.BlockSpec((1, 1, _KP), lambda c: (c, 0, 0))
    cspec = pl.BlockSpec((1, _KP, 1), lambda c: (c, 0, 0))

    s2 = pl.pallas_call(
        _nms_kernel,
        grid=(_NC,),
        in_specs=[rspec, rspec, rspec, rspec, rspec,
                  cspec, cspec, cspec, cspec],
        out_specs=rspec,
        out_shape=jax.ShapeDtypeStruct((_NC, 1, _KP), jnp.float32),
    )(row(sp), row(px1), row(py1), row(px2), row(py2),
      col(px1), col(py1), col(px2), col(py2))

    s2 = s2.reshape(_NC, _KP)
    det_s, det_i = lax.top_k(s2, _NUM_DETECTIONS)
    dx1 = jnp.take_along_axis(px1, det_i, axis=1)
    dy1 = jnp.take_along_axis(py1, det_i, axis=1)
    dx2 = jnp.take_along_axis(px2, det_i, axis=1)
    dy2 = jnp.take_along_axis(py2, det_i, axis=1)
    det_box = jnp.stack([dx1, dy1, dx2, dy2], axis=-1)

    det_valid = det_s > 0.0
    det_box = jnp.where(det_valid[:, :, None], det_box, 0.0)
    det_s = jnp.where(det_valid, det_s, 0.0)
    labels = jnp.broadcast_to(jnp.arange(1, _NUM_CLASSES)[:, None],
                              (_NC, _NUM_DETECTIONS))
    labels = jnp.where(det_valid, labels, 0).astype(jnp.float32)
    det = jnp.concatenate([det_box.reshape(-1, 4),
                           det_s.reshape(-1, 1),
                           labels.reshape(-1, 1)], axis=1)
    return det


# speedup vs baseline: 7.5376x; 1.5230x over previous
import functools
import math

import jax
import jax.numpy as jnp
from jax import lax
from jax.experimental import pallas as pl
from jax.experimental.pallas import tpu as pltpu
from jax.experimental.pallas import tpu_sc as plsc

_N = 20000
_NUM_CLASSES = 21
_NC = _NUM_CLASSES - 1
_IMG_H, _IMG_W = 800.0, 1216.0
_SCORE_THRESH = 0.05
_NMS_THRESH = 0.5
_NUM_DETECTIONS = 100
_MIN_SIZE = 1.0
_K_PRE = 1000
_KP = 1024
_B = 128
_BBOX_XFORM_CLIP = math.log(1000.0 / 16.0)


def _prep_kernel(l_ref, dx_ref, dy_ref, dw_ref, dh_ref, p_ref,
                 s_ref, x1_ref, y1_ref, x2_ref, y2_ref, t_ref, k_ref):
    l = l_ref[...]
    m = jnp.max(l, axis=0, keepdims=True)
    e = jnp.exp(l - m)
    denom = jnp.sum(e, axis=0, keepdims=True)
    score = e[1:, :] / denom

    px1 = p_ref[0:1, :]
    py1 = p_ref[1:2, :]
    px2 = p_ref[2:3, :]
    py2 = p_ref[3:4, :]
    w = px2 - px1
    h = py2 - py1
    cx = px1 + 0.5 * w
    cy = py1 + 0.5 * h

    dx = dx_ref[...] / 10.0
    dy = dy_ref[...] / 10.0
    dw = jnp.minimum(dw_ref[...] / 5.0, _BBOX_XFORM_CLIP)
    dh = jnp.minimum(dh_ref[...] / 5.0, _BBOX_XFORM_CLIP)

    pcx = dx * w + cx
    pcy = dy * h + cy
    pw = jnp.exp(dw) * w
    ph = jnp.exp(dh) * h

    x1 = jnp.clip(pcx - 0.5 * pw, 0.0, _IMG_W)
    y1 = jnp.clip(pcy - 0.5 * ph, 0.0, _IMG_H)
    x2 = jnp.clip(pcx + 0.5 * pw, 0.0, _IMG_W)
    y2 = jnp.clip(pcy + 0.5 * ph, 0.0, _IMG_H)

    bw = x2 - x1
    bh = y2 - y1
    valid = (score >= _SCORE_THRESH) & (bw >= _MIN_SIZE) & (bh >= _MIN_SIZE)
    s = jnp.where(valid, score, -1.0)
    s_ref[...] = s
    x1_ref[...] = x1
    y1_ref[...] = y1
    x2_ref[...] = x2
    y2_ref[...] = y2

    keys = jnp.where(s < 0.0, jnp.int32(-1),
                     lax.bitcast_convert_type(s, jnp.int32))
    k_ref[...] = keys
    lo0 = jnp.full((_NC, 1), -1, jnp.int32)
    hi0 = jnp.full((_NC, 1), 0x3F800001, jnp.int32)

    def bs_body(_, st):
        lo, hi = st
        mid = lo + (hi - lo) // 2
        cnt = jnp.sum((keys >= mid).astype(jnp.int32), axis=1, keepdims=True)
        pred = cnt >= _K_PRE
        return jnp.where(pred, mid, lo), jnp.where(pred, hi, mid)

    lo, hi = lax.fori_loop(0, 31, bs_body, (lo0, hi0))
    t_ref[...] = lo


def _make_compact_sc():
    mesh = plsc.VectorSubcoreMesh(core_axis_name="c", subcore_axis_name="s")

    @functools.partial(
        pl.kernel, mesh=mesh,
        compiler_params=pltpu.CompilerParams(needs_layout_passes=False),
        out_type=[jax.ShapeDtypeStruct((_NC * _KP,), jnp.int32),
                  jax.ShapeDtypeStruct((_NC * _KP,), jnp.int32),
                  jax.ShapeDtypeStruct((_NC * 16,), jnp.int32)],
        scratch_types=[pltpu.VMEM((_N,), jnp.int32),
                       pltpu.VMEM((16,), jnp.int32),
                       pltpu.VMEM((_KP,), jnp.int32),
                       pltpu.VMEM((_KP,), jnp.int32),
                       pltpu.VMEM((16,), jnp.int32)],
    )
    def compact(keys_hbm, thr_hbm, gt_hbm, eq_hbm, cnt_hbm,
                keys_v, thr_v, gt_v, eq_v, cnt_v):
        wid = lax.axis_index("s") * 2 + lax.axis_index("c")
        cls = jnp.minimum(wid, _NC - 1)
        pltpu.sync_copy(keys_hbm.at[pl.ds(cls * _N, _N)], keys_v)
        pltpu.sync_copy(thr_hbm.at[pl.ds(cls * 16, 16)], thr_v)
        thr = thr_v[...]
        lane = lax.iota(jnp.int32, 16)

        def step(i, st):
            off_gt, off_eq = st
            kv = keys_v[pl.ds(i * 16, 16)]
            gidx = lane + i * 16
            m_gt = kv > thr
            m_eq = kv == thr
            pos_gt = off_gt + plsc.cumsum(m_gt.astype(jnp.int32))
            pos_eq = off_eq + plsc.cumsum(m_eq.astype(jnp.int32))
            plsc.store_scatter(gt_v, [pos_gt - 1], gidx,
                               mask=m_gt & (pos_gt <= _KP))
            plsc.store_scatter(eq_v, [pos_eq - 1], gidx,
                               mask=m_eq & (pos_eq <= _KP))
            return (off_gt + plsc.all_reduce_population_count(m_gt),
                    off_eq + plsc.all_reduce_population_count(m_eq))

        zero = jnp.zeros((16,), jnp.int32)
        cnt_gt, cnt_eq = lax.fori_loop(0, _N // 16, step, (zero, zero))
        cnt_v[...] = jnp.where(lane == 0, cnt_gt,
                               jnp.where(lane == 1, cnt_eq, 0))
        pltpu.sync_copy(gt_v, gt_hbm.at[pl.ds(cls * _KP, _KP)])
        pltpu.sync_copy(eq_v, eq_hbm.at[pl.ds(cls * _KP, _KP)])
        pltpu.sync_copy(cnt_v, cnt_hbm.at[pl.ds(cls * 16, 16)])

    return compact


_compact_sc = _make_compact_sc()


def _nms_kernel(s_ref, x1r_ref, y1r_ref, x2r_ref, y2r_ref,
                x1c_ref, y1c_ref, x2c_ref, y2c_ref, s2_ref):
    s_row = s_ref[0]
    x1r = x1r_ref[0]
    y1r = y1r_ref[0]
    x2r = x2r_ref[0]
    y2r = y2r_ref[0]
    x1c = x1c_ref[0]
    y1c = y1c_ref[0]
    x2c = x2c_ref[0]
    y2c = y2c_ref[0]

    area_r = (x2r - x1r) * (y2r - y1r)
    area_c = (x2c - x1c) * (y2c - y1c)
    ltx = jnp.maximum(x1c, x1r)
    lty = jnp.maximum(y1c, y1r)
    rbx = jnp.minimum(x2c, x2r)
    rby = jnp.minimum(y2c, y2r)
    iw = jnp.clip(rbx - ltx, 0.0, None)
    ih = jnp.clip(rby - lty, 0.0, None)
    inter = iw * ih
    union = area_c + area_r - inter
    iou = inter / jnp.maximum(union, 1e-9)
    sup = (iou > _NMS_THRESH).astype(jnp.float32)

    validf = (s_row > 0.0).astype(jnp.float32)

    ii = lax.broadcasted_iota(jnp.int32, (_B, _B), 0)
    jj = lax.broadcasted_iota(jnp.int32, (_B, _B), 1)
    tri = (ii < jj).astype(jnp.float32)

    supp_cnt = jnp.zeros((1, _KP), jnp.float32)
    kept_blocks = []
    for b in range(_KP // _B):
        lo = b * _B
        d = sup[lo:lo + _B, lo:lo + _B] * tri
        vblk = validf[:, lo:lo + _B] * (supp_cnt[:, lo:lo + _B] == 0.0)

        def body(st):
            k, _, it = st
            cnt = jnp.dot(k, d, preferred_element_type=jnp.float32)
            knew = vblk * (cnt == 0.0)
            return knew, k, it + 1

        def cond(st):
            k, prev, it = st
            return jnp.logical_and(it < _B, jnp.any(k != prev))

        k0 = vblk
        kfin, _, _ = lax.while_loop(cond, body, (k0, -jnp.ones_like(k0), 0))
        kept_blocks.append(kfin)
        supp_cnt = supp_cnt + jnp.dot(kfin, sup[lo:lo + _B, :],
                                      preferred_element_type=jnp.float32)

    kept = jnp.concatenate(kept_blocks, axis=1)
    s2_ref[...] = jnp.where(kept > 0.0, s_row, -1.0)[None]


def kernel(class_logit, box_regression, proposal):
    lT = class_logit.T
    deltas = box_regression.reshape(_N, _NUM_CLASSES, 4)[:, 1:, :]
    dT = deltas.transpose(1, 2, 0)
    pT = proposal.T

    out_sd = [jax.ShapeDtypeStruct((_NC, _N), jnp.float32)] * 5 + [
        jax.ShapeDtypeStruct((_NC, 1), jnp.int32),
        jax.ShapeDtypeStruct((_NC, _N), jnp.int32)]
    s, bx1, by1, bx2, by2, tkey, keys = pl.pallas_call(
        _prep_kernel,
        out_shape=out_sd,
    )(lT, dT[:, 0], dT[:, 1], dT[:, 2], dT[:, 3], pT)

    thrb = jnp.broadcast_to(tkey, (_NC, 16)).reshape(-1)
    idx_gt, idx_eq, cnts = _compact_sc(keys.reshape(-1), thrb)
    idx_gt = idx_gt.reshape(_NC, _KP)
    idx_eq = idx_eq.reshape(_NC, _KP)
    cnts = cnts.reshape(_NC, 16)
    cnt_gt = cnts[:, 0:1]
    cnt_eq = cnts[:, 1:2]
    slot = jnp.arange(_KP, dtype=jnp.int32)[None]
    cand_idx = jnp.concatenate([idx_gt, idx_eq], axis=1)
    cand_ok = jnp.concatenate([slot < cnt_gt, slot < cnt_eq], axis=1)
    cand_s = jnp.where(cand_ok,
                       jnp.take_along_axis(s, cand_idx, axis=1, mode="clip"),
                       -2.0)
    neg_sorted, idx_sorted = lax.sort((-cand_s, cand_idx), num_keys=2)
    top_s = -neg_sorted[:, :_K_PRE]
    top_i = idx_sorted[:, :_K_PRE]
    gx1 = jnp.take_along_axis(bx1, top_i, axis=1)
    gy1 = jnp.take_along_axis(by1, top_i, axis=1)
    gx2 = jnp.take_along_axis(bx2, top_i, axis=1)
    gy2 = jnp.take_along_axis(by2, top_i, axis=1)

    pad = _KP - _K_PRE
    sp = jnp.pad(top_s, ((0, 0), (0, pad)), constant_values=-1.0)
    px1 = jnp.pad(gx1, ((0, 0), (0, pad)))
    py1 = jnp.pad(gy1, ((0, 0), (0, pad)))
    px2 = jnp.pad(gx2, ((0, 0), (0, pad)))
    py2 = jnp.pad(gy2, ((0, 0), (0, pad)))

    row = lambda a: a.reshape(_NC, 1, _KP)
    col = lambda a: a.reshape(_NC, _KP, 1)
    rspec = pl.BlockSpec((1, 1, _KP), lambda c: (c, 0, 0))
    cspec = pl.BlockSpec((1, _KP, 1), lambda c: (c, 0, 0))

    s2 = pl.pallas_call(
        _nms_kernel,
        grid=(_NC,),
        in_specs=[rspec, rspec, rspec, rspec, rspec,
                  cspec, cspec, cspec, cspec],
        out_specs=rspec,
        out_shape=jax.ShapeDtypeStruct((_NC, 1, _KP), jnp.float32),
    )(row(sp), row(px1), row(py1), row(px2), row(py2),
      col(px1), col(py1), col(px2), col(py2))

    s2 = s2.reshape(_NC, _KP)
    det_s, det_i = lax.top_k(s2, _NUM_DETECTIONS)
    dx1 = jnp.take_along_axis(px1, det_i, axis=1)
    dy1 = jnp.take_along_axis(py1, det_i, axis=1)
    dx2 = jnp.take_along_axis(px2, det_i, axis=1)
    dy2 = jnp.take_along_axis(py2, det_i, axis=1)
    det_box = jnp.stack([dx1, dy1, dx2, dy2], axis=-1)

    det_valid = det_s > 0.0
    det_box = jnp.where(det_valid[:, :, None], det_box, 0.0)
    det_s = jnp.where(det_valid, det_s, 0.0)
    labels = jnp.broadcast_to(jnp.arange(1, _NUM_CLASSES)[:, None],
                              (_NC, _NUM_DETECTIONS))
    labels = jnp.where(det_valid, labels, 0).astype(jnp.float32)
    det = jnp.concatenate([det_box.reshape(-1, 4),
                           det_s.reshape(-1, 1),
                           labels.reshape(-1, 1)], axis=1)
    return det
